# baseline (speedup 1.0000x reference)
import jax
import jax.numpy as jnp
from jax import lax
from jax.experimental import pallas as pl
from jax.experimental.pallas import tpu as pltpu
from jax.experimental.pallas import tpu_sc as plsc

NC = 2
NS = 16
NW = NC * NS
NBUF = 4


def _pool_body(seq, hidden, bpw, chunk_base, ids_hbm, table_hbm, x_hbm,
               idx_v, rows_bufs, xbuf, sems):
    wid = lax.axis_index("s") * NC + lax.axis_index("c")
    pltpu.sync_copy(ids_hbm.at[pl.ds(chunk_base + wid * bpw, bpw)], idx_v)

    def start_gather(b):
        buf = rows_bufs[b % NBUF]
        sem = sems[b % NBUF]
        w0 = min(128, seq)
        c0 = pltpu.async_copy(table_hbm.at[idx_v.at[b, pl.ds(0, w0)]],
                              buf.at[pl.ds(0, w0)], sem)
        c1 = pltpu.async_copy(table_hbm.at[idx_v.at[b, pl.ds(w0, seq - w0)]],
                              buf.at[pl.ds(w0, seq - w0)], sem)
        return (c0, c1)

    pending = [None] * NBUF
    for b in range(min(NBUF - 1, bpw)):
        pending[b] = start_gather(b)
    scale = jnp.float32(1.0 / seq)
    unroll = 8
    assert seq % unroll == 0
    for b in range(bpw):
        if b + NBUF - 1 < bpw:
            pending[(b + NBUF - 1) % NBUF] = start_gather(b + NBUF - 1)
        for c in pending[b % NBUF]:
            c.wait()
        rows = rows_bufs[b % NBUF]

        def body(i, accs):
            a0, a1, a2, a3 = accs
            s0 = i * unroll
            for u in range(unroll):
                a0 = a0 + rows[s0 + u, pl.ds(0, 16)]
                a1 = a1 + rows[s0 + u, pl.ds(16, 16)]
                a2 = a2 + rows[s0 + u, pl.ds(32, 16)]
                a3 = a3 + rows[s0 + u, pl.ds(48, 16)]
            return (a0, a1, a2, a3)

        z = jnp.zeros((16,), jnp.float32)
        a0, a1, a2, a3 = lax.fori_loop(0, seq // unroll, body, (z, z, z, z))
        xbuf[b, pl.ds(0, 16)] = a0 * scale
        xbuf[b, pl.ds(16, 16)] = a1 * scale
        xbuf[b, pl.ds(32, 16)] = a2 * scale
        xbuf[b, pl.ds(48, 16)] = a3 * scale
    pltpu.sync_copy(xbuf, x_hbm.at[pl.ds(wid * bpw, bpw)])


def _pool_sc(input_ids, table, chunk_base, cb):
    _, seq = input_ids.shape
    hidden = table.shape[1]
    bpw = cb // NW
    mesh = plsc.VectorSubcoreMesh(core_axis_name="c", subcore_axis_name="s")

    def body(ids_hbm, table_hbm, x_hbm, idx_v, *rest):
        rows_bufs = rest[:NBUF]
        xbuf = rest[NBUF]
        sems = rest[NBUF + 1:]
        _pool_body(seq, hidden, bpw, chunk_base, ids_hbm, table_hbm, x_hbm,
                   idx_v, rows_bufs, xbuf, sems)

    fn = pl.kernel(
        body,
        out_type=jax.ShapeDtypeStruct((cb, hidden), jnp.float32),
        mesh=mesh,
        scratch_types=(
            [pltpu.VMEM((bpw, seq), jnp.int32)]
            + [pltpu.VMEM((seq, hidden), jnp.float32) for _ in range(NBUF)]
            + [pltpu.VMEM((bpw, hidden), jnp.float32)]
            + [pltpu.SemaphoreType.DMA for _ in range(NBUF)]
        ),
        compiler_params=pltpu.CompilerParams(use_tc_tiling_on_sc=False),
    )
    return fn(input_ids, table)


def _mm_body(x_ref, w_ref, b_ref, out_ref):
    out_ref[...] = lax.dot_general(
        x_ref[...], w_ref[...],
        dimension_numbers=(((1,), (1,)), ((), ())),
        preferred_element_type=jnp.float32,
    ) + b_ref[...]


def _mm_body_aliased(x_ref, w_ref, b_ref, _buf_ref, out_ref):
    _mm_body(x_ref, w_ref, b_ref, out_ref)


def _linear_chunk(x, W, b2, buf, chunk, batch, rb=512, vb=8192):
    cb, hidden = x.shape
    vocab = W.shape[0]
    grid = (cb // rb, pl.cdiv(vocab, vb))
    in_specs = [
        pl.BlockSpec((rb, hidden), lambda i, j: (i, 0)),
        pl.BlockSpec((vb, hidden), lambda i, j: (j, 0)),
        pl.BlockSpec((1, vb), lambda i, j: (0, j)),
    ]
    args = [x, W, b2]
    kwargs = {}
    body = _mm_body
    if buf is not None:
        in_specs.append(pl.BlockSpec(memory_space=pl.ANY))
        args.append(buf)
        kwargs["input_output_aliases"] = {3: 0}
        body = _mm_body_aliased
    nb = cb // rb
    return pl.pallas_call(
        body,
        grid=grid,
        in_specs=in_specs,
        out_specs=pl.BlockSpec(
            (rb, vb), lambda i, j, c=chunk, n=nb: (c * n + i, j)),
        out_shape=jax.ShapeDtypeStruct((batch, vocab), jnp.float32),
        compiler_params=pltpu.CompilerParams(
            dimension_semantics=("parallel", "parallel")),
        **kwargs,
    )(*args)


def kernel(input_ids, table, W, b, nchunk=1):
    batch, _ = input_ids.shape
    vocab = W.shape[0]
    cb = batch // nchunk
    b2 = b.reshape(1, vocab)
    xs = []
    logits = None
    for i in range(nchunk):
        xs.append(_pool_sc(input_ids, table, i * cb, cb))
        logits = _linear_chunk(xs[i], W, b2, logits, i, batch)
    x = jnp.concatenate(xs, axis=0) if nchunk > 1 else xs[0]
    return (logits, x)


# speedup vs baseline: 1.0253x; 1.0253x over previous
import jax
import jax.numpy as jnp
from jax import lax
from jax.experimental import pallas as pl
from jax.experimental.pallas import tpu as pltpu
from jax.experimental.pallas import tpu_sc as plsc

NC = 2
NS = 16
NW = NC * NS
NBUF = 4


def _pool_body(seq, hidden, bpw, chunk_base, ids_hbm, table_hbm, x_hbm,
               idx_v, rows_bufs, xbuf, sems):
    wid = lax.axis_index("s") * NC + lax.axis_index("c")
    pltpu.sync_copy(ids_hbm.at[pl.ds(chunk_base + wid * bpw, bpw)], idx_v)

    def start_gather(b):
        buf = rows_bufs[b % NBUF]
        sem = sems[b % NBUF]
        w0 = min(128, seq)
        c0 = pltpu.async_copy(table_hbm.at[idx_v.at[b, pl.ds(0, w0)]],
                              buf.at[pl.ds(0, w0)], sem)
        c1 = pltpu.async_copy(table_hbm.at[idx_v.at[b, pl.ds(w0, seq - w0)]],
                              buf.at[pl.ds(w0, seq - w0)], sem)
        return (c0, c1)

    pending = [None] * NBUF
    for b in range(min(NBUF - 1, bpw)):
        pending[b] = start_gather(b)
    scale = jnp.float32(1.0 / seq)
    unroll = 8
    assert seq % unroll == 0
    for b in range(bpw):
        if b + NBUF - 1 < bpw:
            pending[(b + NBUF - 1) % NBUF] = start_gather(b + NBUF - 1)
        for c in pending[b % NBUF]:
            c.wait()
        rows = rows_bufs[b % NBUF]

        def body(i, accs):
            a0, a1, a2, a3 = accs
            s0 = i * unroll
            for u in range(unroll):
                a0 = a0 + rows[s0 + u, pl.ds(0, 16)]
                a1 = a1 + rows[s0 + u, pl.ds(16, 16)]
                a2 = a2 + rows[s0 + u, pl.ds(32, 16)]
                a3 = a3 + rows[s0 + u, pl.ds(48, 16)]
            return (a0, a1, a2, a3)

        z = jnp.zeros((16,), jnp.float32)
        a0, a1, a2, a3 = lax.fori_loop(0, seq // unroll, body, (z, z, z, z))
        xbuf[b, pl.ds(0, 16)] = a0 * scale
        xbuf[b, pl.ds(16, 16)] = a1 * scale
        xbuf[b, pl.ds(32, 16)] = a2 * scale
        xbuf[b, pl.ds(48, 16)] = a3 * scale
    pltpu.sync_copy(xbuf, x_hbm.at[pl.ds(wid * bpw, bpw)])


def _pool_sc(input_ids, table, chunk_base, cb):
    _, seq = input_ids.shape
    hidden = table.shape[1]
    bpw = cb // NW
    mesh = plsc.VectorSubcoreMesh(core_axis_name="c", subcore_axis_name="s")

    def body(ids_hbm, table_hbm, x_hbm, idx_v, *rest):
        rows_bufs = rest[:NBUF]
        xbuf = rest[NBUF]
        sems = rest[NBUF + 1:]
        _pool_body(seq, hidden, bpw, chunk_base, ids_hbm, table_hbm, x_hbm,
                   idx_v, rows_bufs, xbuf, sems)

    fn = pl.kernel(
        body,
        out_type=jax.ShapeDtypeStruct((cb, hidden), jnp.float32),
        mesh=mesh,
        scratch_types=(
            [pltpu.VMEM((bpw, seq), jnp.int32)]
            + [pltpu.VMEM((seq, hidden), jnp.float32) for _ in range(NBUF)]
            + [pltpu.VMEM((bpw, hidden), jnp.float32)]
            + [pltpu.SemaphoreType.DMA for _ in range(NBUF)]
        ),
        compiler_params=pltpu.CompilerParams(use_tc_tiling_on_sc=False),
    )
    return fn(input_ids, table)


def _mm_body(x_ref, w_ref, b_ref, out_ref):
    out_ref[...] = lax.dot_general(
        x_ref[...], w_ref[...],
        dimension_numbers=(((1,), (1,)), ((), ())),
        preferred_element_type=jnp.float32,
    ) + b_ref[...]


def _mm_body_aliased(x_ref, w_ref, b_ref, _buf_ref, out_ref):
    _mm_body(x_ref, w_ref, b_ref, out_ref)


def _linear_chunk(x, W, b2, buf, chunk, batch, rb=1024, vb=6144):
    cb, hidden = x.shape
    vocab = W.shape[0]
    grid = (cb // rb, pl.cdiv(vocab, vb))
    in_specs = [
        pl.BlockSpec((rb, hidden), lambda i, j: (i, 0)),
        pl.BlockSpec((vb, hidden), lambda i, j: (j, 0)),
        pl.BlockSpec((1, vb), lambda i, j: (0, j)),
    ]
    args = [x, W, b2]
    kwargs = {}
    body = _mm_body
    if buf is not None:
        in_specs.append(pl.BlockSpec(memory_space=pl.ANY))
        args.append(buf)
        kwargs["input_output_aliases"] = {3: 0}
        body = _mm_body_aliased
    nb = cb // rb
    return pl.pallas_call(
        body,
        grid=grid,
        in_specs=in_specs,
        out_specs=pl.BlockSpec(
            (rb, vb), lambda i, j, c=chunk, n=nb: (c * n + i, j)),
        out_shape=jax.ShapeDtypeStruct((batch, vocab), jnp.float32),
        compiler_params=pltpu.CompilerParams(
            dimension_semantics=("parallel", "parallel")),
        **kwargs,
    )(*args)


def kernel(input_ids, table, W, b, nchunk=1):
    batch, _ = input_ids.shape
    vocab = W.shape[0]
    cb = batch // nchunk
    b2 = b.reshape(1, vocab)
    xs = []
    logits = None
    for i in range(nchunk):
        xs.append(_pool_sc(input_ids, table, i * cb, cb))
        logits = _linear_chunk(xs[i], W, b2, logits, i, batch)
    x = jnp.concatenate(xs, axis=0) if nchunk > 1 else xs[0]
    return (logits, x)


# speedup vs baseline: 1.0259x; 1.0006x over previous
import jax
import jax.numpy as jnp
from jax import lax
from jax.experimental import pallas as pl
from jax.experimental.pallas import tpu as pltpu
from jax.experimental.pallas import tpu_sc as plsc

NC = 2
NS = 16
NW = NC * NS
NBUF = 6


def _pool_body(seq, hidden, bpw, chunk_base, ids_hbm, table_hbm, x_hbm,
               idx_v, rows_bufs, xbuf, sems):
    wid = lax.axis_index("s") * NC + lax.axis_index("c")
    pltpu.sync_copy(ids_hbm.at[pl.ds(chunk_base + wid * bpw, bpw)], idx_v)

    def start_gather(b):
        buf = rows_bufs[b % NBUF]
        sem = sems[b % NBUF]
        w0 = min(128, seq)
        c0 = pltpu.async_copy(table_hbm.at[idx_v.at[b, pl.ds(0, w0)]],
                              buf.at[pl.ds(0, w0)], sem)
        c1 = pltpu.async_copy(table_hbm.at[idx_v.at[b, pl.ds(w0, seq - w0)]],
                              buf.at[pl.ds(w0, seq - w0)], sem)
        return (c0, c1)

    pending = [None] * NBUF
    for b in range(min(NBUF - 1, bpw)):
        pending[b] = start_gather(b)
    scale = jnp.float32(1.0 / seq)
    unroll = 8
    assert seq % unroll == 0
    for b in range(bpw):
        if b + NBUF - 1 < bpw:
            pending[(b + NBUF - 1) % NBUF] = start_gather(b + NBUF - 1)
        for c in pending[b % NBUF]:
            c.wait()
        rows = rows_bufs[b % NBUF]

        def body(i, accs):
            a0, a1, a2, a3 = accs
            s0 = i * unroll
            for u in range(unroll):
                a0 = a0 + rows[s0 + u, pl.ds(0, 16)]
                a1 = a1 + rows[s0 + u, pl.ds(16, 16)]
                a2 = a2 + rows[s0 + u, pl.ds(32, 16)]
                a3 = a3 + rows[s0 + u, pl.ds(48, 16)]
            return (a0, a1, a2, a3)

        z = jnp.zeros((16,), jnp.float32)
        a0, a1, a2, a3 = lax.fori_loop(0, seq // unroll, body, (z, z, z, z))
        xbuf[b, pl.ds(0, 16)] = a0 * scale
        xbuf[b, pl.ds(16, 16)] = a1 * scale
        xbuf[b, pl.ds(32, 16)] = a2 * scale
        xbuf[b, pl.ds(48, 16)] = a3 * scale
    pltpu.sync_copy(xbuf, x_hbm.at[pl.ds(wid * bpw, bpw)])


def _pool_sc(input_ids, table, chunk_base, cb):
    _, seq = input_ids.shape
    hidden = table.shape[1]
    bpw = cb // NW
    mesh = plsc.VectorSubcoreMesh(core_axis_name="c", subcore_axis_name="s")

    def body(ids_hbm, table_hbm, x_hbm, idx_v, *rest):
        rows_bufs = rest[:NBUF]
        xbuf = rest[NBUF]
        sems = rest[NBUF + 1:]
        _pool_body(seq, hidden, bpw, chunk_base, ids_hbm, table_hbm, x_hbm,
                   idx_v, rows_bufs, xbuf, sems)

    fn = pl.kernel(
        body,
        out_type=jax.ShapeDtypeStruct((cb, hidden), jnp.float32),
        mesh=mesh,
        scratch_types=(
            [pltpu.VMEM((bpw, seq), jnp.int32)]
            + [pltpu.VMEM((seq, hidden), jnp.float32) for _ in range(NBUF)]
            + [pltpu.VMEM((bpw, hidden), jnp.float32)]
            + [pltpu.SemaphoreType.DMA for _ in range(NBUF)]
        ),
        compiler_params=pltpu.CompilerParams(use_tc_tiling_on_sc=False),
    )
    return fn(input_ids, table)


def _mm_body(x_ref, w_ref, b_ref, out_ref):
    out_ref[...] = lax.dot_general(
        x_ref[...], w_ref[...],
        dimension_numbers=(((1,), (1,)), ((), ())),
        preferred_element_type=jnp.float32,
    ) + b_ref[...]


def _mm_body_aliased(x_ref, w_ref, b_ref, _buf_ref, out_ref):
    _mm_body(x_ref, w_ref, b_ref, out_ref)


def _linear_chunk(x, W, b2, buf, chunk, batch, rb=1024, vb=6144):
    cb, hidden = x.shape
    vocab = W.shape[0]
    grid = (cb // rb, pl.cdiv(vocab, vb))
    in_specs = [
        pl.BlockSpec((rb, hidden), lambda i, j: (i, 0)),
        pl.BlockSpec((vb, hidden), lambda i, j: (j, 0)),
        pl.BlockSpec((1, vb), lambda i, j: (0, j)),
    ]
    args = [x, W, b2]
    kwargs = {}
    body = _mm_body
    if buf is not None:
        in_specs.append(pl.BlockSpec(memory_space=pl.ANY))
        args.append(buf)
        kwargs["input_output_aliases"] = {3: 0}
        body = _mm_body_aliased
    nb = cb // rb
    return pl.pallas_call(
        body,
        grid=grid,
        in_specs=in_specs,
        out_specs=pl.BlockSpec(
            (rb, vb), lambda i, j, c=chunk, n=nb: (c * n + i, j)),
        out_shape=jax.ShapeDtypeStruct((batch, vocab), jnp.float32),
        compiler_params=pltpu.CompilerParams(
            dimension_semantics=("parallel", "parallel")),
        **kwargs,
    )(*args)


def kernel(input_ids, table, W, b, nchunk=1):
    batch, _ = input_ids.shape
    vocab = W.shape[0]
    cb = batch // nchunk
    b2 = b.reshape(1, vocab)
    xs = []
    logits = None
    for i in range(nchunk):
        xs.append(_pool_sc(input_ids, table, i * cb, cb))
        logits = _linear_chunk(xs[i], W, b2, logits, i, batch)
    x = jnp.concatenate(xs, axis=0) if nchunk > 1 else xs[0]
    return (logits, x)


# speedup vs baseline: 1.0277x; 1.0017x over previous
import jax
import jax.numpy as jnp
from jax import lax
from jax.experimental import pallas as pl
from jax.experimental.pallas import tpu as pltpu
from jax.experimental.pallas import tpu_sc as plsc

NC = 2
NS = 16
NW = NC * NS
NBUF = 4


def _pool_body(seq, hidden, bpw, chunk_base, ids_hbm, table_hbm, x_hbm,
               idx_v, rows_bufs, xbuf, sems):
    wid = lax.axis_index("s") * NC + lax.axis_index("c")
    pltpu.sync_copy(ids_hbm.at[pl.ds(chunk_base + wid * bpw, bpw)], idx_v)

    def start_gather(b):
        buf = rows_bufs[b % NBUF]
        sem = sems[b % NBUF]
        w0 = min(128, seq)
        c0 = pltpu.async_copy(table_hbm.at[idx_v.at[b, pl.ds(0, w0)]],
                              buf.at[pl.ds(0, w0)], sem)
        c1 = pltpu.async_copy(table_hbm.at[idx_v.at[b, pl.ds(w0, seq - w0)]],
                              buf.at[pl.ds(w0, seq - w0)], sem)
        return (c0, c1)

    pending = [None] * NBUF
    for b in range(min(NBUF - 1, bpw)):
        pending[b] = start_gather(b)
    scale = jnp.float32(1.0 / seq)
    unroll = 8
    assert seq % unroll == 0
    for b in range(bpw):
        if b + NBUF - 1 < bpw:
            pending[(b + NBUF - 1) % NBUF] = start_gather(b + NBUF - 1)
        for c in pending[b % NBUF]:
            c.wait()
        rows = rows_bufs[b % NBUF]

        def body(i, accs):
            a0, a1, a2, a3 = accs
            s0 = i * unroll
            for u in range(unroll):
                a0 = a0 + rows[s0 + u, pl.ds(0, 16)]
                a1 = a1 + rows[s0 + u, pl.ds(16, 16)]
                a2 = a2 + rows[s0 + u, pl.ds(32, 16)]
                a3 = a3 + rows[s0 + u, pl.ds(48, 16)]
            return (a0, a1, a2, a3)

        z = jnp.zeros((16,), jnp.float32)
        a0, a1, a2, a3 = lax.fori_loop(0, seq // unroll, body, (z, z, z, z))
        xbuf[b, pl.ds(0, 16)] = a0 * scale
        xbuf[b, pl.ds(16, 16)] = a1 * scale
        xbuf[b, pl.ds(32, 16)] = a2 * scale
        xbuf[b, pl.ds(48, 16)] = a3 * scale
    pltpu.sync_copy(xbuf, x_hbm.at[pl.ds(wid * bpw, bpw)])


def _pool_sc(input_ids, table, chunk_base, cb):
    _, seq = input_ids.shape
    hidden = table.shape[1]
    bpw = cb // NW
    mesh = plsc.VectorSubcoreMesh(core_axis_name="c", subcore_axis_name="s")

    def body(ids_hbm, table_hbm, x_hbm, idx_v, *rest):
        rows_bufs = rest[:NBUF]
        xbuf = rest[NBUF]
        sems = rest[NBUF + 1:]
        _pool_body(seq, hidden, bpw, chunk_base, ids_hbm, table_hbm, x_hbm,
                   idx_v, rows_bufs, xbuf, sems)

    fn = pl.kernel(
        body,
        out_type=jax.ShapeDtypeStruct((cb, hidden), jnp.float32),
        mesh=mesh,
        scratch_types=(
            [pltpu.VMEM((bpw, seq), jnp.int32)]
            + [pltpu.VMEM((seq, hidden), jnp.float32) for _ in range(NBUF)]
            + [pltpu.VMEM((bpw, hidden), jnp.float32)]
            + [pltpu.SemaphoreType.DMA for _ in range(NBUF)]
        ),
        compiler_params=pltpu.CompilerParams(use_tc_tiling_on_sc=False),
    )
    return fn(input_ids, table)


def _mm_body(x_ref, w_ref, b_ref, out_ref):
    out_ref[...] = lax.dot_general(
        x_ref[...], w_ref[...],
        dimension_numbers=(((1,), (1,)), ((), ())),
        preferred_element_type=jnp.float32,
    ) + b_ref[...]


def _mm_body_aliased(x_ref, w_ref, b_ref, _buf_ref, out_ref):
    _mm_body(x_ref, w_ref, b_ref, out_ref)


def _linear_chunk(x, W, b2, buf, chunk, batch, rb=1024, vb=6144):
    cb, hidden = x.shape
    vocab = W.shape[0]
    grid = (cb // rb, pl.cdiv(vocab, vb))
    in_specs = [
        pl.BlockSpec((rb, hidden), lambda i, j: (i, 0)),
        pl.BlockSpec((vb, hidden), lambda i, j: (j, 0)),
        pl.BlockSpec((1, vb), lambda i, j: (0, j)),
    ]
    args = [x, W, b2]
    kwargs = {}
    body = _mm_body
    if buf is not None:
        in_specs.append(pl.BlockSpec(memory_space=pl.ANY))
        args.append(buf)
        kwargs["input_output_aliases"] = {3: 0}
        body = _mm_body_aliased
    nb = cb // rb
    return pl.pallas_call(
        body,
        grid=grid,
        in_specs=in_specs,
        out_specs=pl.BlockSpec(
            (rb, vb), lambda i, j, c=chunk, n=nb: (c * n + i, j)),
        out_shape=jax.ShapeDtypeStruct((batch, vocab), jnp.float32),
        compiler_params=pltpu.CompilerParams(
            dimension_semantics=("parallel", "parallel")),
        **kwargs,
    )(*args)


def kernel(input_ids, table, W, b, nchunk=1):
    batch, _ = input_ids.shape
    vocab = W.shape[0]
    cb = batch // nchunk
    b2 = b.reshape(1, vocab)
    xs = []
    logits = None
    for i in range(nchunk):
        xs.append(_pool_sc(input_ids, table, i * cb, cb))
        logits = _linear_chunk(xs[i], W, b2, logits, i, batch)
    x = jnp.concatenate(xs, axis=0) if nchunk > 1 else xs[0]
    return (logits, x)
